# Initial kernel scaffold; baseline (speedup 1.0000x reference)
import math

import jax
import jax.numpy as jnp
from jax import lax
from jax.experimental import pallas as pl
from jax.experimental.pallas import tpu as pltpu
from jax.experimental.pallas import tpu_sc as plsc

N_LEVELS = 6
BASE_RES = 16
DESIRED_RES = 256
N_FEAT = 4
N_POINTS = 524288

_log2_scale = math.log2(DESIRED_RES / BASE_RES) / (N_LEVELS - 1)
RES_L = [int(math.ceil(2.0 ** (l * _log2_scale) * BASE_RES - 1.0)) + 1
         for l in range(N_LEVELS)]
ENT_OFF = [0]
for _r in RES_L:
    ENT_OFF.append(ENT_OFF[-1] + _r * _r)
TOTAL_ENT = ENT_OFF[-1]
ENT_OFF = ENT_OFF[:-1]

NC, NS = 2, 16
NW = NC * NS
PW = N_POINTS // NW
C = 128
G = C // 16
NCHUNK = PW // C
NSLOT = 2 * N_LEVELS


def _quad_pack(p):
    outs = []
    for l in range(N_LEVELS):
        r, o = RES_L[l], ENT_OFF[l]
        s = p[o:o + r * r].reshape(r, r, N_FEAT)
        sx = jnp.concatenate([s[:, 1:], s[:, -1:]], axis=1)
        sy = jnp.concatenate([s[1:], s[-1:]], axis=0)
        sxy = jnp.concatenate([sy[:, 1:], sy[:, -1:]], axis=1)
        outs.append(jnp.concatenate([s, sx, sy, sxy], axis=-1).reshape(r * r, 16))
    return jnp.concatenate(outs, axis=0)


def _body(c1_hbm, c2_hbm, t_hbm, qt_hbm, out_hbm,
          cb1, cb2, cbt, idxb, wb, db, ost, sem):
    wid = lax.axis_index("s") * NC + lax.axis_index("c")
    iota = lax.iota(jnp.int32, 16)

    def chunk_body(i, carry):
        base = wid * PW + i * C
        pltpu.sync_copy(c1_hbm.at[pl.ds(base, C)], cb1)
        pltpu.sync_copy(c2_hbm.at[pl.ds(base, C)], cb2)
        pltpu.sync_copy(t_hbm.at[pl.ds(base, C)], cbt)

        def pass_a(g, cc):
            r0 = g * 16
            tc = jnp.minimum(jnp.maximum(cbt[pl.ds(r0, 16)], 0.0), 1.0)
            cv = [jnp.minimum(jnp.maximum(cb2[pl.ds(r0, 16)], 0.0), 1.0),
                  jnp.minimum(jnp.maximum(cb1[pl.ds(r0, 16)], 0.0), 1.0)]
            for l in range(N_LEVELS):
                res = RES_L[l]
                pt = tc * float(res - 1)
                iy = jnp.minimum(pt.astype(jnp.int32), res - 2)
                wy = pt - iy.astype(jnp.float32)
                omy = 1.0 - wy
                for p in range(2):
                    s = p * N_LEVELS + l
                    px = cv[p] * float(res - 1)
                    ix = jnp.minimum(px.astype(jnp.int32), res - 2)
                    wx = px - ix.astype(jnp.float32)
                    omx = 1.0 - wx
                    idxb[s, pl.ds(r0, 16)] = iy * res + ix + (p * TOTAL_ENT + ENT_OFF[l])
                    wb[s * 4 + 0, pl.ds(r0, 16)] = omx * omy
                    wb[s * 4 + 1, pl.ds(r0, 16)] = wx * omy
                    wb[s * 4 + 2, pl.ds(r0, 16)] = omx * wy
                    wb[s * 4 + 3, pl.ds(r0, 16)] = wx * wy
            return cc
        lax.fori_loop(0, G, pass_a, 0)

        cps = [pltpu.async_copy(qt_hbm.at[idxb.at[s]],
                                db.at[pl.ds(s * C, C)], sem)
               for s in range(NSLOT)]
        for cp in cps:
            cp.wait()

        def pass_c(g, cc):
            r0 = g * 16
            orow = r0 + iota
            for s in range(NSLOT):
                p, l = s // N_LEVELS, s % N_LEVELS
                rowv = s * C + r0 + iota
                q = [plsc.load_gather(db, [rowv, jnp.full((16,), c, jnp.int32)])
                     for c in range(16)]
                w0 = wb[s * 4 + 0, pl.ds(r0, 16)]
                w1 = wb[s * 4 + 1, pl.ds(r0, 16)]
                w2 = wb[s * 4 + 2, pl.ds(r0, 16)]
                w3 = wb[s * 4 + 3, pl.ds(r0, 16)]
                for f in range(4):
                    o = q[f] * w0 + q[4 + f] * w1 + q[8 + f] * w2 + q[12 + f] * w3
                    if p == 0:
                        plsc.store_scatter(ost, [orow, jnp.full((16,), l * 4 + f, jnp.int32)], o)
                        plsc.store_scatter(ost, [orow, jnp.full((16,), 48 + l * 4 + f, jnp.int32)], o)
                    else:
                        plsc.store_scatter(ost, [orow, jnp.full((16,), 24 + l * 4 + f, jnp.int32)], o)
            return cc
        lax.fori_loop(0, G, pass_c, 0)

        pltpu.sync_copy(ost, out_hbm.at[pl.ds(base, C)])
        return carry

    lax.fori_loop(0, NCHUNK, chunk_body, 0)


def kernel(x, time, bound, params_xt, params_yt, params_zt):
    del params_zt
    n = x.shape[0]
    xn = (x + bound) / (2 * bound)
    c1 = xn[:, 1]
    c2 = xn[:, 2]
    tt = time.reshape(n)
    qt = jnp.concatenate([_quad_pack(params_xt), _quad_pack(params_yt)], axis=0)

    mesh = plsc.VectorSubcoreMesh(core_axis_name="c", subcore_axis_name="s",
                                  num_cores=NC, num_subcores=NS)
    run = pl.kernel(
        _body,
        out_type=jax.ShapeDtypeStruct((n, 72), jnp.float32),
        mesh=mesh,
        scratch_types=[
            pltpu.VMEM((C,), jnp.float32),
            pltpu.VMEM((C,), jnp.float32),
            pltpu.VMEM((C,), jnp.float32),
            pltpu.VMEM((NSLOT, C), jnp.int32),
            pltpu.VMEM((NSLOT * 4, C), jnp.float32),
            pltpu.VMEM((NSLOT * C, 16), jnp.float32),
            pltpu.VMEM((C, 72), jnp.float32),
            pltpu.SemaphoreType.DMA,
        ],
    )
    return run(c1, c2, tt, qt)


# speedup vs baseline: 43.4487x; 43.4487x over previous
import math

import jax
import jax.numpy as jnp
from jax import lax
from jax.experimental import pallas as pl
from jax.experimental.pallas import tpu as pltpu
from jax.experimental.pallas import tpu_sc as plsc

N_LEVELS = 6
BASE_RES = 16
DESIRED_RES = 256
N_FEAT = 4
N_POINTS = 524288

_log2_scale = math.log2(DESIRED_RES / BASE_RES) / (N_LEVELS - 1)
RES_L = [int(math.ceil(2.0 ** (l * _log2_scale) * BASE_RES - 1.0)) + 1
         for l in range(N_LEVELS)]
ENT_OFF = [0]
for _r in RES_L:
    ENT_OFF.append(ENT_OFF[-1] + _r * _r)
TOTAL_ENT = ENT_OFF[-1]
ENT_OFF = ENT_OFF[:-1]

NC, NS = 2, 16
NW = NC * NS
PW = N_POINTS // NW
C = 128
G = C // 16
NCHUNK = PW // C
NSLOT = 2 * N_LEVELS


def _quad_pack(p):
    outs = []
    for l in range(N_LEVELS):
        r, o = RES_L[l], ENT_OFF[l]
        s = p[o:o + r * r].reshape(r, r, N_FEAT)
        sx = jnp.concatenate([s[:, 1:], s[:, -1:]], axis=1)
        sy = jnp.concatenate([s[1:], s[-1:]], axis=0)
        sxy = jnp.concatenate([sy[:, 1:], sy[:, -1:]], axis=1)
        outs.append(jnp.concatenate([s, sx, sy, sxy], axis=-1).reshape(r * r, 16))
    return jnp.concatenate(outs, axis=0)


def _body(c1_hbm, c2_hbm, t_hbm, qt_hbm, out_hbm,
          cb1, cb2, cbt, idxb, wb, db, ost, sem):
    wid = lax.axis_index("s") * NC + lax.axis_index("c")
    iota = lax.iota(jnp.int32, 16)

    def chunk_body(i, carry):
        base = wid * PW + i * C
        pltpu.sync_copy(c1_hbm.at[pl.ds(base, C)], cb1)
        pltpu.sync_copy(c2_hbm.at[pl.ds(base, C)], cb2)
        pltpu.sync_copy(t_hbm.at[pl.ds(base, C)], cbt)

        def pass_a(g, cc):
            r0 = g * 16
            tc = jnp.minimum(jnp.maximum(cbt[pl.ds(r0, 16)], 0.0), 1.0)
            cv = [jnp.minimum(jnp.maximum(cb2[pl.ds(r0, 16)], 0.0), 1.0),
                  jnp.minimum(jnp.maximum(cb1[pl.ds(r0, 16)], 0.0), 1.0)]
            for l in range(N_LEVELS):
                res = RES_L[l]
                pt = tc * float(res - 1)
                iy = jnp.minimum(pt.astype(jnp.int32), res - 2)
                wy = pt - iy.astype(jnp.float32)
                omy = 1.0 - wy
                for p in range(2):
                    s = p * N_LEVELS + l
                    px = cv[p] * float(res - 1)
                    ix = jnp.minimum(px.astype(jnp.int32), res - 2)
                    wx = px - ix.astype(jnp.float32)
                    omx = 1.0 - wx
                    idxb[s, pl.ds(r0, 16)] = iy * res + ix + (p * TOTAL_ENT + ENT_OFF[l])
                    wb[s * 4 + 0, pl.ds(r0, 16)] = omx * omy
                    wb[s * 4 + 1, pl.ds(r0, 16)] = wx * omy
                    wb[s * 4 + 2, pl.ds(r0, 16)] = omx * wy
                    wb[s * 4 + 3, pl.ds(r0, 16)] = wx * wy
            return cc
        lax.fori_loop(0, G, pass_a, 0)

        cps = [pltpu.async_copy(qt_hbm.at[idxb.at[s]],
                                db.at[pl.ds(s * C, C)], sem)
               for s in range(NSLOT)]
        for cp in cps:
            cp.wait()

        def pass_c(g, cc):
            r0 = g * 16
            orow = r0 + iota
            for s in range(NSLOT):
                p, l = s // N_LEVELS, s % N_LEVELS
                rowv = s * C + r0 + iota
                q = [plsc.load_gather(db, [rowv, jnp.full((16,), c, jnp.int32)])
                     for c in range(16)]
                w0 = wb[s * 4 + 0, pl.ds(r0, 16)]
                w1 = wb[s * 4 + 1, pl.ds(r0, 16)]
                w2 = wb[s * 4 + 2, pl.ds(r0, 16)]
                w3 = wb[s * 4 + 3, pl.ds(r0, 16)]
                for f in range(4):
                    o = q[f] * w0 + q[4 + f] * w1 + q[8 + f] * w2 + q[12 + f] * w3
                    if p == 0:
                        plsc.store_scatter(ost, [orow, jnp.full((16,), l * 4 + f, jnp.int32)], o)
                        plsc.store_scatter(ost, [orow, jnp.full((16,), 48 + l * 4 + f, jnp.int32)], o)
                    else:
                        plsc.store_scatter(ost, [orow, jnp.full((16,), 24 + l * 4 + f, jnp.int32)], o)
            return cc
        lax.fori_loop(0, G, pass_c, 0)

        pltpu.sync_copy(ost, out_hbm.at[pl.ds(base, C)])
        return carry

    lax.fori_loop(0, NCHUNK, chunk_body, 0)


def kernel(x, time, bound, params_xt, params_yt, params_zt):
    del params_zt
    n = x.shape[0]
    xn = (x + bound) / (2 * bound)
    c1 = xn[:, 1]
    c2 = xn[:, 2]
    tt = time.reshape(n)
    qt = jnp.concatenate([_quad_pack(params_xt), _quad_pack(params_yt)], axis=0)

    mesh = plsc.VectorSubcoreMesh(core_axis_name="c", subcore_axis_name="s",
                                  num_cores=NC, num_subcores=NS)
    run = pl.kernel(
        _body,
        out_type=jax.ShapeDtypeStruct((n, 72), jnp.float32),
        mesh=mesh,
        compiler_params=pltpu.CompilerParams(needs_layout_passes=False,
                                             use_tc_tiling_on_sc=False),
        scratch_types=[
            pltpu.VMEM((C,), jnp.float32),
            pltpu.VMEM((C,), jnp.float32),
            pltpu.VMEM((C,), jnp.float32),
            pltpu.VMEM((NSLOT, C), jnp.int32),
            pltpu.VMEM((NSLOT * 4, C), jnp.float32),
            pltpu.VMEM((NSLOT * C, 16), jnp.float32),
            pltpu.VMEM((C, 72), jnp.float32),
            pltpu.SemaphoreType.DMA,
        ],
    )
    return run(c1, c2, tt, qt)
